# baseline (speedup 1.0000x reference)
import jax
import jax.numpy as jnp
from jax import lax
from jax.experimental import pallas as pl
from jax.experimental.pallas import tpu as pltpu
from jax.experimental.pallas import tpu_sc as plsc

N = 10000
E = 320000
D = 128
NG = 8
ROWS_G = N // NG
OUT_G = 64
CHUNK = 128
NC = 2
NS = 16
NW = NC * NS
ROWS_T = N // NS
ZROWS = 125
ITERS = -(-E // (CHUNK * NW))
NCHP = ITERS * NW
EPAD = NCHP * CHUNK
NPAD = N + 8
NB = 3
MAIN = (ITERS // NB) * NB
TCB = 2000
TCG = N // TCB

_MESH = plsc.VectorSubcoreMesh(core_axis_name="c", subcore_axis_name="s")
_SC_PARAMS = pltpu.CompilerParams(use_tc_tiling_on_sc=False)


def _hist_body(sr_ref, out_s, out_r,
               srb0, srb1, srb2, srb3,
               ones_v, zbuf, accs, accr, i0, i1, i2, i3, q0, q1, q2, q3):
    cid = lax.axis_index("c")
    sid = lax.axis_index("s")
    wid = sid * NC + cid
    srb = [srb0, srb1, srb2, srb3]
    isem = [i0, i1, i2, i3]
    qsem = [q0, q1, q2, q3]

    zero16 = jnp.zeros((16,), jnp.float32)
    one16 = jnp.ones((16,), jnp.float32)

    @pl.loop(0, ROWS_T)
    def _(i):
        zbuf[i, :] = zero16

    @pl.loop(0, CHUNK)
    def _(i):
        ones_v[i, :] = one16

    sl = pl.ds(sid * ROWS_T, ROWS_T)
    pltpu.sync_copy(zbuf, accs.at[sl])
    pltpu.sync_copy(zbuf, accr.at[sl])
    plsc.subcore_barrier()

    def process(jbase, count):
        idescs = []
        for k in range(count):
            c = (jbase + k) * NW + wid
            idescs.append(pltpu.async_copy(sr_ref.at[c], srb[k], isem[k]))
        sdescs = []
        for k in range(count):
            idescs[k].wait()
            sdescs.append(pltpu.async_copy(ones_v, accs.at[srb[k].at[0]],
                                           qsem[k], add=True))
            sdescs.append(pltpu.async_copy(ones_v, accr.at[srb[k].at[1]],
                                           qsem[k], add=True))
        for d in sdescs:
            d.wait()

    @pl.loop(0, MAIN, step=NB)
    def _(J):
        process(J, NB)

    if ITERS > MAIN:
        process(MAIN, ITERS - MAIN)

    plsc.subcore_barrier()
    pltpu.sync_copy(accs.at[sl], out_s.at[cid, sid])
    pltpu.sync_copy(accr.at[sl], out_r.at[cid, sid])


def _sc_hist(sr):
    kern = pl.kernel(
        _hist_body,
        out_type=(jax.ShapeDtypeStruct((NC, NS, ROWS_T, 16), jnp.float32),
                  jax.ShapeDtypeStruct((NC, NS, ROWS_T, 16), jnp.float32)),
        mesh=_MESH,
        scratch_types=(
            [pltpu.VMEM((2, CHUNK), jnp.int32)] * 4
            + [pltpu.VMEM((CHUNK, 16), jnp.float32),
               pltpu.VMEM((ROWS_T, 16), jnp.float32),
               pltpu.VMEM_SHARED((NPAD, 16), jnp.float32),
               pltpu.VMEM_SHARED((NPAD, 16), jnp.float32)]
            + [pltpu.SemaphoreType.DMA] * 8),
        compiler_params=_SC_PARAMS,
    )
    hs, hr = kern(sr)
    return hs.reshape(NC, N, 16), hr.reshape(NC, N, 16)


def _conv_body(xs_ref, sr_ref, out_ref,
               srb0, srb1, srb2,
               rows0, rows1, rows2, acc,
               g0, g1, g2, s0, s1, s2s, i0, i1, i2):
    cid = lax.axis_index("c")
    sid = lax.axis_index("s")
    wid = sid * NC + cid
    srb = [srb0, srb1, srb2]
    rows = [rows0, rows1, rows2]
    g = [g0, g1, g2]
    s = [s0, s1, s2s]
    isem = [i0, i1, i2]

    zero16 = jnp.zeros((16,), jnp.float32)
    for lg in range(D // 16):
        @pl.loop(0, ZROWS)
        def _(i):
            rows0[i, lg * 16:(lg + 1) * 16] = zero16

    @pl.loop(0, ROWS_T // ZROWS)
    def _(k):
        pltpu.sync_copy(rows0.at[pl.ds(0, ZROWS)],
                        acc.at[pl.ds(sid * ROWS_T + k * ZROWS, ZROWS)])

    plsc.subcore_barrier()

    @pl.loop(0, ITERS)
    def _(j):
        c = j * NW + wid
        pltpu.async_copy(sr_ref.at[c], srb[0], isem[0]).wait()
        pltpu.async_copy(xs_ref.at[srb[0].at[0]], rows[0], g[0]).wait()
        pltpu.async_copy(rows[0], acc.at[srb[0].at[1]], s[0],
                         add=True).wait()

    plsc.subcore_barrier()
    sl = pl.ds(sid * ROWS_T, ROWS_T)
    pltpu.sync_copy(acc.at[sl], out_ref.at[cid, sid])


def _sc_conv(xs, sr):
    kern = pl.kernel(
        _conv_body,
        out_type=jax.ShapeDtypeStruct((NC, NS, ROWS_T, D), jnp.float32),
        mesh=_MESH,
        scratch_types=(
            [pltpu.VMEM((2, CHUNK), jnp.int32)] * 3
            + [pltpu.VMEM((CHUNK, D), jnp.float32)] * 3
            + [pltpu.VMEM_SHARED((NPAD, D), jnp.float32)]
            + [pltpu.SemaphoreType.DMA] * 9),
        compiler_params=_SC_PARAMS,
    )
    return kern(xs, sr).reshape(NC, N, D)


def _embed_body(nodes_ref, we_ref, be_ref, edges_ref, h_ref, e4_ref):
    h_ref[...] = (jnp.dot(nodes_ref[...], we_ref[...],
                          preferred_element_type=jnp.float32) + be_ref[...])
    e4_ref[...] = edges_ref[...] * 4.0


def _tc_embed(nodes, W_embed, b_embed, e2):
    return pl.pallas_call(
        _embed_body,
        grid=(TCG,),
        in_specs=[
            pl.BlockSpec((TCB, D), lambda i: (i, 0)),
            pl.BlockSpec((D, D), lambda i: (0, 0)),
            pl.BlockSpec((D,), lambda i: (0,)),
            pl.BlockSpec((TCB, D), lambda i: (i, 0)),
        ],
        out_specs=[
            pl.BlockSpec((TCB, D), lambda i: (i, 0)),
            pl.BlockSpec((TCB, D), lambda i: (i, 0)),
        ],
        out_shape=[jax.ShapeDtypeStruct((N, D), jnp.float32),
                   jax.ShapeDtypeStruct((N, D), jnp.float32)],
    )(nodes, W_embed, b_embed, e2)


def _mlp_body(h_ref, w0_ref, b0_ref, w1_ref, b1_ref, hs_ref, xs_ref):
    x = jnp.maximum(jnp.dot(h_ref[...], w0_ref[...],
                            preferred_element_type=jnp.float32) + b0_ref[...],
                    0.0)
    x = jnp.maximum(jnp.dot(x, w1_ref[...],
                            preferred_element_type=jnp.float32) + b1_ref[...],
                    0.0)
    hs = hs_ref[...]
    sdeg = hs[0, :, 0:1] + hs[1, :, 0:1] + 1.0
    xs_ref[...] = x * lax.rsqrt(sdeg)


def _tc_mlp(h, w0, b0, w1, b1, hist_s):
    return pl.pallas_call(
        _mlp_body,
        grid=(TCG,),
        in_specs=[
            pl.BlockSpec((TCB, D), lambda i: (i, 0)),
            pl.BlockSpec((D, D), lambda i: (0, 0)),
            pl.BlockSpec((D,), lambda i: (0,)),
            pl.BlockSpec((D, D), lambda i: (0, 0)),
            pl.BlockSpec((D,), lambda i: (0,)),
            pl.BlockSpec((NC, TCB, 16), lambda i: (0, i, 0)),
        ],
        out_specs=pl.BlockSpec((TCB, D), lambda i: (i, 0)),
        out_shape=jax.ShapeDtypeStruct((N, D), jnp.float32),
    )(h, w0, b0, w1, b1, hist_s)


def _update_body(acc_ref, xs_ref, h_ref, hr_ref, sc_ref, bi_ref, out_ref):
    hr = hr_ref[...]
    rdeg = hr[0, :, 0:1] + hr[1, :, 0:1] + 1.0
    acc = acc_ref[...]
    xs = xs_ref[...]
    t = (acc[0] + acc[1] + xs) * lax.rsqrt(rdeg) + h_ref[...]
    m = jnp.mean(t, axis=-1, keepdims=True)
    v = jnp.mean(jnp.square(t - m), axis=-1, keepdims=True)
    out_ref[...] = ((t - m) * lax.rsqrt(v + 1e-6)) * sc_ref[...] + bi_ref[...]


def _tc_update(acc, xs, h, hist_r, lns, lnb):
    return pl.pallas_call(
        _update_body,
        grid=(TCG,),
        in_specs=[
            pl.BlockSpec((NC, TCB, D), lambda i: (0, i, 0)),
            pl.BlockSpec((TCB, D), lambda i: (i, 0)),
            pl.BlockSpec((TCB, D), lambda i: (i, 0)),
            pl.BlockSpec((NC, TCB, 16), lambda i: (0, i, 0)),
            pl.BlockSpec((D,), lambda i: (0,)),
            pl.BlockSpec((D,), lambda i: (0,)),
        ],
        out_specs=pl.BlockSpec((TCB, D), lambda i: (i, 0)),
        out_shape=jax.ShapeDtypeStruct((N, D), jnp.float32),
    )(acc, xs, h, hist_r, lns, lnb)


def _decode_body(h_ref, wd_ref, bd_ref, out_ref):
    col_graph = lax.broadcasted_iota(jnp.int32, (NG, N), 1) // ROWS_G
    row_id = lax.broadcasted_iota(jnp.int32, (NG, N), 0)
    gsel = (col_graph == row_id).astype(jnp.float32)
    pooled = jnp.dot(gsel, h_ref[...],
                     preferred_element_type=jnp.float32) * (1.0 / ROWS_G)
    out_ref[...] = (jnp.dot(pooled, wd_ref[...],
                            preferred_element_type=jnp.float32) + bd_ref[...])


def _tc_decode(h, W_dec, b_dec):
    return pl.pallas_call(
        _decode_body,
        out_shape=jax.ShapeDtypeStruct((NG, OUT_G), jnp.float32),
    )(h, W_dec, b_dec)


def kernel(nodes, edges, senders, receivers, globals_, n_node, n_edge,
           W_embed, b_embed,
           W_s0_l0, b_s0_l0, W_s0_l1, b_s0_l1, ln0_scale, ln0_bias,
           W_s1_l0, b_s1_l0, W_s1_l1, b_s1_l1, ln1_scale, ln1_bias,
           W_dec, b_dec):
    pad_dump = jnp.full((EPAD - E,), N, dtype=senders.dtype)
    pad_zero = jnp.zeros((EPAD - E,), dtype=senders.dtype)
    s2h = jnp.concatenate([senders, pad_dump]).reshape(NCHP, CHUNK)
    s2c = jnp.concatenate([senders, pad_zero]).reshape(NCHP, CHUNK)
    r2 = jnp.concatenate([receivers, pad_dump]).reshape(NCHP, CHUNK)
    srh = jnp.stack([s2h, r2], axis=1)
    src = jnp.stack([s2c, r2], axis=1)
    e2 = edges.reshape(N, D)

    hist_s, hist_r = _sc_hist(srh)
    h, e4 = _tc_embed(nodes, W_embed, b_embed, e2)

    steps = [
        (W_s0_l0, b_s0_l0, W_s0_l1, b_s0_l1, ln0_scale, ln0_bias),
        (W_s1_l0, b_s1_l0, W_s1_l1, b_s1_l1, ln1_scale, ln1_bias),
    ]
    for w0, b0, w1, b1, lns, lnb in steps:
        xs = _tc_mlp(h, w0, b0, w1, b1, hist_s)
        acc = _sc_conv(xs, src)
        h = _tc_update(acc, xs, h, hist_r, lns, lnb)

    out_globals = _tc_decode(h, W_dec, b_dec)
    return h, e4.reshape(E, 4), out_globals


# speedup vs baseline: 1.0017x; 1.0017x over previous
import jax
import jax.numpy as jnp
from jax import lax
from jax.experimental import pallas as pl
from jax.experimental.pallas import tpu as pltpu
from jax.experimental.pallas import tpu_sc as plsc

N = 10000
E = 320000
D = 128
NG = 8
ROWS_G = N // NG
OUT_G = 64
CHUNK = 128
NC = 2
NS = 16
NW = NC * NS
ROWS_T = N // NS
ZROWS = 125
ITERS = -(-E // (CHUNK * NW))
NCHP = ITERS * NW
EPAD = NCHP * CHUNK
NPAD = N + 8
NB = 3
MAIN = (ITERS // NB) * NB
TCB = 2000
TCG = N // TCB

_MESH = plsc.VectorSubcoreMesh(core_axis_name="c", subcore_axis_name="s")
_SC_PARAMS = pltpu.CompilerParams(use_tc_tiling_on_sc=False)


def _hist_body(sr_ref, out_s, out_r,
               srb0, srb1, srb2, srb3,
               ones_v, zbuf, accs, accr, i0, i1, i2, i3, q0, q1, q2, q3):
    cid = lax.axis_index("c")
    sid = lax.axis_index("s")
    wid = sid * NC + cid
    srb = [srb0, srb1, srb2, srb3]
    isem = [i0, i1, i2, i3]
    qsem = [q0, q1, q2, q3]

    zero16 = jnp.zeros((16,), jnp.float32)
    one16 = jnp.ones((16,), jnp.float32)

    @pl.loop(0, ROWS_T)
    def _(i):
        zbuf[i, :] = zero16

    @pl.loop(0, CHUNK)
    def _(i):
        ones_v[i, :] = one16

    sl = pl.ds(sid * ROWS_T, ROWS_T)
    pltpu.sync_copy(zbuf, accs.at[sl])
    pltpu.sync_copy(zbuf, accr.at[sl])
    plsc.subcore_barrier()

    def process(jbase, count):
        idescs = []
        for k in range(count):
            c = (jbase + k) * NW + wid
            idescs.append(pltpu.async_copy(sr_ref.at[c], srb[k], isem[k]))
        sdescs = []
        for k in range(count):
            idescs[k].wait()
            sdescs.append(pltpu.async_copy(ones_v, accs.at[srb[k].at[0]],
                                           qsem[k], add=True))
            sdescs.append(pltpu.async_copy(ones_v, accr.at[srb[k].at[1]],
                                           qsem[k], add=True))
        for d in sdescs:
            d.wait()

    @pl.loop(0, MAIN, step=NB)
    def _(J):
        process(J, NB)

    if ITERS > MAIN:
        process(MAIN, ITERS - MAIN)

    plsc.subcore_barrier()
    pltpu.sync_copy(accs.at[sl], out_s.at[cid, sid])
    pltpu.sync_copy(accr.at[sl], out_r.at[cid, sid])


def _sc_hist(sr):
    kern = pl.kernel(
        _hist_body,
        out_type=(jax.ShapeDtypeStruct((NC, NS, ROWS_T, 16), jnp.float32),
                  jax.ShapeDtypeStruct((NC, NS, ROWS_T, 16), jnp.float32)),
        mesh=_MESH,
        scratch_types=(
            [pltpu.VMEM((2, CHUNK), jnp.int32)] * 4
            + [pltpu.VMEM((CHUNK, 16), jnp.float32),
               pltpu.VMEM((ROWS_T, 16), jnp.float32),
               pltpu.VMEM_SHARED((NPAD, 16), jnp.float32),
               pltpu.VMEM_SHARED((NPAD, 16), jnp.float32)]
            + [pltpu.SemaphoreType.DMA] * 8),
        compiler_params=_SC_PARAMS,
    )
    hs, hr = kern(sr)
    return hs.reshape(NC, N, 16), hr.reshape(NC, N, 16)


def _conv_body(xs_ref, sr_ref, out_ref,
               srb0, srb1, srb2,
               rows0, rows1, rows2, acc,
               g0, g1, g2, s0, s1, s2s, i0, i1, i2):
    cid = lax.axis_index("c")
    sid = lax.axis_index("s")
    wid = sid * NC + cid
    srb = [srb0, srb1, srb2]
    rows = [rows0, rows1, rows2]
    g = [g0, g1, g2]
    s = [s0, s1, s2s]
    isem = [i0, i1, i2]

    zero16 = jnp.zeros((16,), jnp.float32)
    for lg in range(D // 16):
        @pl.loop(0, ZROWS)
        def _(i):
            rows0[i, lg * 16:(lg + 1) * 16] = zero16

    @pl.loop(0, ROWS_T // ZROWS)
    def _(k):
        pltpu.sync_copy(rows0.at[pl.ds(0, ZROWS)],
                        acc.at[pl.ds(sid * ROWS_T + k * ZROWS, ZROWS)])

    plsc.subcore_barrier()

    @pl.loop(0, ITERS)
    def _(j):
        c = j * NW + wid
        pltpu.sync_copy(sr_ref.at[c], srb[0])
        pltpu.async_copy(xs_ref.at[srb[0].at[0]], rows[0], g[0]).wait()
        pltpu.sync_copy(rows[0], acc.at[srb[0].at[1]], add=True)

    plsc.subcore_barrier()
    sl = pl.ds(sid * ROWS_T, ROWS_T)
    pltpu.sync_copy(acc.at[sl], out_ref.at[cid, sid])


def _sc_conv(xs, sr):
    kern = pl.kernel(
        _conv_body,
        out_type=jax.ShapeDtypeStruct((NC, NS, ROWS_T, D), jnp.float32),
        mesh=_MESH,
        scratch_types=(
            [pltpu.VMEM((2, CHUNK), jnp.int32)] * 3
            + [pltpu.VMEM((CHUNK, D), jnp.float32)] * 3
            + [pltpu.VMEM_SHARED((NPAD, D), jnp.float32)]
            + [pltpu.SemaphoreType.DMA] * 9),
        compiler_params=_SC_PARAMS,
    )
    return kern(xs, sr).reshape(NC, N, D)


def _embed_body(nodes_ref, we_ref, be_ref, edges_ref, h_ref, e4_ref):
    h_ref[...] = (jnp.dot(nodes_ref[...], we_ref[...],
                          preferred_element_type=jnp.float32) + be_ref[...])
    e4_ref[...] = edges_ref[...] * 4.0


def _tc_embed(nodes, W_embed, b_embed, e2):
    return pl.pallas_call(
        _embed_body,
        grid=(TCG,),
        in_specs=[
            pl.BlockSpec((TCB, D), lambda i: (i, 0)),
            pl.BlockSpec((D, D), lambda i: (0, 0)),
            pl.BlockSpec((D,), lambda i: (0,)),
            pl.BlockSpec((TCB, D), lambda i: (i, 0)),
        ],
        out_specs=[
            pl.BlockSpec((TCB, D), lambda i: (i, 0)),
            pl.BlockSpec((TCB, D), lambda i: (i, 0)),
        ],
        out_shape=[jax.ShapeDtypeStruct((N, D), jnp.float32),
                   jax.ShapeDtypeStruct((N, D), jnp.float32)],
    )(nodes, W_embed, b_embed, e2)


def _mlp_body(h_ref, w0_ref, b0_ref, w1_ref, b1_ref, hs_ref, xs_ref):
    x = jnp.maximum(jnp.dot(h_ref[...], w0_ref[...],
                            preferred_element_type=jnp.float32) + b0_ref[...],
                    0.0)
    x = jnp.maximum(jnp.dot(x, w1_ref[...],
                            preferred_element_type=jnp.float32) + b1_ref[...],
                    0.0)
    hs = hs_ref[...]
    sdeg = hs[0, :, 0:1] + hs[1, :, 0:1] + 1.0
    xs_ref[...] = x * lax.rsqrt(sdeg)


def _tc_mlp(h, w0, b0, w1, b1, hist_s):
    return pl.pallas_call(
        _mlp_body,
        grid=(TCG,),
        in_specs=[
            pl.BlockSpec((TCB, D), lambda i: (i, 0)),
            pl.BlockSpec((D, D), lambda i: (0, 0)),
            pl.BlockSpec((D,), lambda i: (0,)),
            pl.BlockSpec((D, D), lambda i: (0, 0)),
            pl.BlockSpec((D,), lambda i: (0,)),
            pl.BlockSpec((NC, TCB, 16), lambda i: (0, i, 0)),
        ],
        out_specs=pl.BlockSpec((TCB, D), lambda i: (i, 0)),
        out_shape=jax.ShapeDtypeStruct((N, D), jnp.float32),
    )(h, w0, b0, w1, b1, hist_s)


def _update_body(acc_ref, xs_ref, h_ref, hr_ref, sc_ref, bi_ref, out_ref):
    hr = hr_ref[...]
    rdeg = hr[0, :, 0:1] + hr[1, :, 0:1] + 1.0
    acc = acc_ref[...]
    xs = xs_ref[...]
    t = (acc[0] + acc[1] + xs) * lax.rsqrt(rdeg) + h_ref[...]
    m = jnp.mean(t, axis=-1, keepdims=True)
    v = jnp.mean(jnp.square(t - m), axis=-1, keepdims=True)
    out_ref[...] = ((t - m) * lax.rsqrt(v + 1e-6)) * sc_ref[...] + bi_ref[...]


def _tc_update(acc, xs, h, hist_r, lns, lnb):
    return pl.pallas_call(
        _update_body,
        grid=(TCG,),
        in_specs=[
            pl.BlockSpec((NC, TCB, D), lambda i: (0, i, 0)),
            pl.BlockSpec((TCB, D), lambda i: (i, 0)),
            pl.BlockSpec((TCB, D), lambda i: (i, 0)),
            pl.BlockSpec((NC, TCB, 16), lambda i: (0, i, 0)),
            pl.BlockSpec((D,), lambda i: (0,)),
            pl.BlockSpec((D,), lambda i: (0,)),
        ],
        out_specs=pl.BlockSpec((TCB, D), lambda i: (i, 0)),
        out_shape=jax.ShapeDtypeStruct((N, D), jnp.float32),
    )(acc, xs, h, hist_r, lns, lnb)


def _decode_body(h_ref, wd_ref, bd_ref, out_ref):
    col_graph = lax.broadcasted_iota(jnp.int32, (NG, N), 1) // ROWS_G
    row_id = lax.broadcasted_iota(jnp.int32, (NG, N), 0)
    gsel = (col_graph == row_id).astype(jnp.float32)
    pooled = jnp.dot(gsel, h_ref[...],
                     preferred_element_type=jnp.float32) * (1.0 / ROWS_G)
    out_ref[...] = (jnp.dot(pooled, wd_ref[...],
                            preferred_element_type=jnp.float32) + bd_ref[...])


def _tc_decode(h, W_dec, b_dec):
    return pl.pallas_call(
        _decode_body,
        out_shape=jax.ShapeDtypeStruct((NG, OUT_G), jnp.float32),
    )(h, W_dec, b_dec)


def kernel(nodes, edges, senders, receivers, globals_, n_node, n_edge,
           W_embed, b_embed,
           W_s0_l0, b_s0_l0, W_s0_l1, b_s0_l1, ln0_scale, ln0_bias,
           W_s1_l0, b_s1_l0, W_s1_l1, b_s1_l1, ln1_scale, ln1_bias,
           W_dec, b_dec):
    pad_dump = jnp.full((EPAD - E,), N, dtype=senders.dtype)
    pad_zero = jnp.zeros((EPAD - E,), dtype=senders.dtype)
    s2h = jnp.concatenate([senders, pad_dump]).reshape(NCHP, CHUNK)
    s2c = jnp.concatenate([senders, pad_zero]).reshape(NCHP, CHUNK)
    r2 = jnp.concatenate([receivers, pad_dump]).reshape(NCHP, CHUNK)
    srh = jnp.stack([s2h, r2], axis=1)
    src = jnp.stack([s2c, r2], axis=1)
    e2 = edges.reshape(N, D)

    hist_s, hist_r = _sc_hist(srh)
    h, e4 = _tc_embed(nodes, W_embed, b_embed, e2)

    steps = [
        (W_s0_l0, b_s0_l0, W_s0_l1, b_s0_l1, ln0_scale, ln0_bias),
        (W_s1_l0, b_s1_l0, W_s1_l1, b_s1_l1, ln1_scale, ln1_bias),
    ]
    for w0, b0, w1, b1, lns, lnb in steps:
        xs = _tc_mlp(h, w0, b0, w1, b1, hist_s)
        acc = _sc_conv(xs, src)
        h = _tc_update(acc, xs, h, hist_r, lns, lnb)

    out_globals = _tc_decode(h, W_dec, b_dec)
    return h, e4.reshape(E, 4), out_globals


# speedup vs baseline: 1.2485x; 1.2463x over previous
import jax
import jax.numpy as jnp
from jax import lax
from jax.experimental import pallas as pl
from jax.experimental.pallas import tpu as pltpu
from jax.experimental.pallas import tpu_sc as plsc

N = 10000
E = 320000
D = 128
NG = 8
ROWS_G = N // NG
OUT_G = 64
CHUNK = 128
NC = 2
NS = 16
NW = NC * NS
ROWS_T = N // NS
ZROWS = 125
NCH = E // CHUNK
ITERS = -(-NCH // NW)
NPAD = N + 8
NB = 3
MAIN = ((ITERS - 1) // NB) * NB
TCB = 2000
TCG = N // TCB

_MESH = plsc.VectorSubcoreMesh(core_axis_name="c", subcore_axis_name="s")
_SC_PARAMS = pltpu.CompilerParams(use_tc_tiling_on_sc=False)


def _hist_body(s_ref, r_ref, out_s, out_r,
               sb0, sb1, sb2, sb3, rb0, rb1, rb2, rb3,
               ones_v, zbuf, accs, accr, i0, i1, i2, i3, q0, q1, q2, q3):
    cid = lax.axis_index("c")
    sid = lax.axis_index("s")
    wid = sid * NC + cid
    sbuf = [sb0, sb1, sb2, sb3]
    rbuf = [rb0, rb1, rb2, rb3]
    isem = [i0, i1, i2, i3]
    qsem = [q0, q1, q2, q3]

    zero16 = jnp.zeros((16,), jnp.float32)
    one16 = jnp.ones((16,), jnp.float32)

    @pl.loop(0, ROWS_T)
    def _(i):
        zbuf[i, :] = zero16

    @pl.loop(0, CHUNK)
    def _(i):
        ones_v[i, :] = one16

    sl = pl.ds(sid * ROWS_T, ROWS_T)
    pltpu.sync_copy(zbuf, accs.at[sl])
    pltpu.sync_copy(zbuf, accr.at[sl])
    plsc.subcore_barrier()

    def process(jbase, count):
        idescs = []
        for k in range(count):
            c = (jbase + k) * NW + wid
            d1 = pltpu.async_copy(s_ref.at[c], sbuf[k], isem[k])
            d2 = pltpu.async_copy(r_ref.at[c], rbuf[k], isem[k])
            idescs.append((d1, d2))
        sdescs = []
        for k in range(count):
            idescs[k][0].wait()
            idescs[k][1].wait()
            sdescs.append(pltpu.async_copy(ones_v, accs.at[sbuf[k].at[0]],
                                           qsem[k], add=True))
            sdescs.append(pltpu.async_copy(ones_v, accr.at[rbuf[k].at[0]],
                                           qsem[k], add=True))
        for d in sdescs:
            d.wait()

    @pl.loop(0, MAIN, step=NB)
    def _(J):
        process(J, NB)

    c_tail = MAIN * NW + wid

    @pl.when(c_tail < NCH)
    def _():
        pltpu.sync_copy(s_ref.at[c_tail], sbuf[0])
        pltpu.sync_copy(r_ref.at[c_tail], rbuf[0])
        pltpu.sync_copy(ones_v, accs.at[sbuf[0].at[0]], add=True)
        pltpu.sync_copy(ones_v, accr.at[rbuf[0].at[0]], add=True)

    plsc.subcore_barrier()
    pltpu.sync_copy(accs.at[sl], out_s.at[cid, sid])
    pltpu.sync_copy(accr.at[sl], out_r.at[cid, sid])


def _sc_hist(s2, r2):
    kern = pl.kernel(
        _hist_body,
        out_type=(jax.ShapeDtypeStruct((NC, NS, ROWS_T, 16), jnp.float32),
                  jax.ShapeDtypeStruct((NC, NS, ROWS_T, 16), jnp.float32)),
        mesh=_MESH,
        scratch_types=(
            [pltpu.VMEM((1, CHUNK), jnp.int32)] * 8
            + [pltpu.VMEM((CHUNK, 16), jnp.float32),
               pltpu.VMEM((ROWS_T, 16), jnp.float32),
               pltpu.VMEM_SHARED((NPAD, 16), jnp.float32),
               pltpu.VMEM_SHARED((NPAD, 16), jnp.float32)]
            + [pltpu.SemaphoreType.DMA] * 8),
        compiler_params=_SC_PARAMS,
    )
    hs, hr = kern(s2, r2)
    return hs.reshape(NC, N, 16), hr.reshape(NC, N, 16)


def _conv_body(xs_ref, s_ref, r_ref, out_ref,
               sidx_v, ridx_v, rows0, acc, g0):
    cid = lax.axis_index("c")
    sid = lax.axis_index("s")
    wid = sid * NC + cid

    zero16 = jnp.zeros((16,), jnp.float32)
    for lg in range(D // 16):
        @pl.loop(0, ZROWS)
        def _(i):
            rows0[i, lg * 16:(lg + 1) * 16] = zero16

    @pl.loop(0, ROWS_T // ZROWS)
    def _(kz):
        pltpu.sync_copy(rows0.at[pl.ds(0, ZROWS)],
                        acc.at[pl.ds(sid * ROWS_T + kz * ZROWS, ZROWS)])

    plsc.subcore_barrier()

    def chunk(c):
        pltpu.sync_copy(s_ref.at[c], sidx_v)
        pltpu.async_copy(xs_ref.at[sidx_v.at[0]], rows0, g0).wait()
        pltpu.sync_copy(r_ref.at[c], ridx_v)
        pltpu.sync_copy(rows0, acc.at[ridx_v.at[0]], add=True)

    @pl.loop(0, MAIN)
    def _(j):
        chunk(j * NW + wid)

    c_tail = MAIN * NW + wid

    @pl.when(c_tail < NCH)
    def _():
        chunk(c_tail)

    plsc.subcore_barrier()
    sl = pl.ds(sid * ROWS_T, ROWS_T)
    pltpu.sync_copy(acc.at[sl], out_ref.at[cid, sid])


def _sc_conv(xs, s2, r2):
    kern = pl.kernel(
        _conv_body,
        out_type=jax.ShapeDtypeStruct((NC, NS, ROWS_T, D), jnp.float32),
        mesh=_MESH,
        scratch_types=[
            pltpu.VMEM((1, CHUNK), jnp.int32),
            pltpu.VMEM((1, CHUNK), jnp.int32),
            pltpu.VMEM((CHUNK, D), jnp.float32),
            pltpu.VMEM_SHARED((NPAD, D), jnp.float32),
            pltpu.SemaphoreType.DMA,
        ],
        compiler_params=_SC_PARAMS,
    )
    return kern(xs, s2, r2).reshape(NC, N, D)


def _embed_body(nodes_ref, we_ref, be_ref, edges_ref, h_ref, e4_ref):
    h_ref[...] = (jnp.dot(nodes_ref[...], we_ref[...],
                          preferred_element_type=jnp.float32) + be_ref[...])
    e4_ref[...] = edges_ref[...] * 4.0


def _tc_embed(nodes, W_embed, b_embed, e2):
    return pl.pallas_call(
        _embed_body,
        grid=(TCG,),
        in_specs=[
            pl.BlockSpec((TCB, D), lambda i: (i, 0)),
            pl.BlockSpec((D, D), lambda i: (0, 0)),
            pl.BlockSpec((D,), lambda i: (0,)),
            pl.BlockSpec((TCB, D), lambda i: (i, 0)),
        ],
        out_specs=[
            pl.BlockSpec((TCB, D), lambda i: (i, 0)),
            pl.BlockSpec((TCB, D), lambda i: (i, 0)),
        ],
        out_shape=[jax.ShapeDtypeStruct((N, D), jnp.float32),
                   jax.ShapeDtypeStruct((N, D), jnp.float32)],
    )(nodes, W_embed, b_embed, e2)


def _mlp_body(h_ref, w0_ref, b0_ref, w1_ref, b1_ref, hs_ref, xs_ref):
    x = jnp.maximum(jnp.dot(h_ref[...], w0_ref[...],
                            preferred_element_type=jnp.float32) + b0_ref[...],
                    0.0)
    x = jnp.maximum(jnp.dot(x, w1_ref[...],
                            preferred_element_type=jnp.float32) + b1_ref[...],
                    0.0)
    hs = hs_ref[...]
    sdeg = hs[0, :, 0:1] + hs[1, :, 0:1] + 1.0
    xs_ref[...] = x * lax.rsqrt(sdeg)


def _tc_mlp(h, w0, b0, w1, b1, hist_s):
    return pl.pallas_call(
        _mlp_body,
        grid=(TCG,),
        in_specs=[
            pl.BlockSpec((TCB, D), lambda i: (i, 0)),
            pl.BlockSpec((D, D), lambda i: (0, 0)),
            pl.BlockSpec((D,), lambda i: (0,)),
            pl.BlockSpec((D, D), lambda i: (0, 0)),
            pl.BlockSpec((D,), lambda i: (0,)),
            pl.BlockSpec((NC, TCB, 16), lambda i: (0, i, 0)),
        ],
        out_specs=pl.BlockSpec((TCB, D), lambda i: (i, 0)),
        out_shape=jax.ShapeDtypeStruct((N, D), jnp.float32),
    )(h, w0, b0, w1, b1, hist_s)


def _update_body(acc_ref, xs_ref, h_ref, hr_ref, sc_ref, bi_ref, out_ref):
    hr = hr_ref[...]
    rdeg = hr[0, :, 0:1] + hr[1, :, 0:1] + 1.0
    acc = acc_ref[...]
    xs = xs_ref[...]
    t = (acc[0] + acc[1] + xs) * lax.rsqrt(rdeg) + h_ref[...]
    m = jnp.mean(t, axis=-1, keepdims=True)
    v = jnp.mean(jnp.square(t - m), axis=-1, keepdims=True)
    out_ref[...] = ((t - m) * lax.rsqrt(v + 1e-6)) * sc_ref[...] + bi_ref[...]


def _tc_update(acc, xs, h, hist_r, lns, lnb):
    return pl.pallas_call(
        _update_body,
        grid=(TCG,),
        in_specs=[
            pl.BlockSpec((NC, TCB, D), lambda i: (0, i, 0)),
            pl.BlockSpec((TCB, D), lambda i: (i, 0)),
            pl.BlockSpec((TCB, D), lambda i: (i, 0)),
            pl.BlockSpec((NC, TCB, 16), lambda i: (0, i, 0)),
            pl.BlockSpec((D,), lambda i: (0,)),
            pl.BlockSpec((D,), lambda i: (0,)),
        ],
        out_specs=pl.BlockSpec((TCB, D), lambda i: (i, 0)),
        out_shape=jax.ShapeDtypeStruct((N, D), jnp.float32),
    )(acc, xs, h, hist_r, lns, lnb)


def _decode_body(h_ref, wd_ref, bd_ref, out_ref):
    col_graph = lax.broadcasted_iota(jnp.int32, (NG, N), 1) // ROWS_G
    row_id = lax.broadcasted_iota(jnp.int32, (NG, N), 0)
    gsel = (col_graph == row_id).astype(jnp.float32)
    pooled = jnp.dot(gsel, h_ref[...],
                     preferred_element_type=jnp.float32) * (1.0 / ROWS_G)
    out_ref[...] = (jnp.dot(pooled, wd_ref[...],
                            preferred_element_type=jnp.float32) + bd_ref[...])


def _tc_decode(h, W_dec, b_dec):
    return pl.pallas_call(
        _decode_body,
        out_shape=jax.ShapeDtypeStruct((NG, OUT_G), jnp.float32),
    )(h, W_dec, b_dec)


def kernel(nodes, edges, senders, receivers, globals_, n_node, n_edge,
           W_embed, b_embed,
           W_s0_l0, b_s0_l0, W_s0_l1, b_s0_l1, ln0_scale, ln0_bias,
           W_s1_l0, b_s1_l0, W_s1_l1, b_s1_l1, ln1_scale, ln1_bias,
           W_dec, b_dec):
    s2 = senders.reshape(NCH, 1, CHUNK)
    r2 = receivers.reshape(NCH, 1, CHUNK)
    e2 = edges.reshape(N, D)

    hist_s, hist_r = _sc_hist(s2, r2)
    h, e4 = _tc_embed(nodes, W_embed, b_embed, e2)

    steps = [
        (W_s0_l0, b_s0_l0, W_s0_l1, b_s0_l1, ln0_scale, ln0_bias),
        (W_s1_l0, b_s1_l0, W_s1_l1, b_s1_l1, ln1_scale, ln1_bias),
    ]
    for w0, b0, w1, b1, lns, lnb in steps:
        xs = _tc_mlp(h, w0, b0, w1, b1, hist_s)
        acc = _sc_conv(xs, s2, r2)
        h = _tc_update(acc, xs, h, hist_r, lns, lnb)

    out_globals = _tc_decode(h, W_dec, b_dec)
    return h, e4.reshape(E, 4), out_globals


# speedup vs baseline: 1.5196x; 1.2171x over previous
import jax
import jax.numpy as jnp
from jax import lax
from jax.experimental import pallas as pl
from jax.experimental.pallas import tpu as pltpu
from jax.experimental.pallas import tpu_sc as plsc

N = 10000
E = 320000
D = 128
NG = 8
ROWS_G = N // NG
OUT_G = 64
CHUNK = 128
NC = 2
NS = 16
NW = NC * NS
ROWS_T = N // NS
ZROWS = 125
NCH = E // CHUNK
ITERS = -(-NCH // NW)
NPAD = N + 8
NB = 3
MAIN = ((ITERS - 1) // NB) * NB
TCB = 2000
TCG = N // TCB

_MESH = plsc.VectorSubcoreMesh(core_axis_name="c", subcore_axis_name="s")
_SC_PARAMS = pltpu.CompilerParams(use_tc_tiling_on_sc=False)


def _hist_body(s_ref, r_ref, out_s, out_r,
               sb0, sb1, sb2, sb3, rb0, rb1, rb2, rb3,
               ones_v, zbuf, accs, accr, i0, i1, i2, i3, q0, q1, q2, q3):
    cid = lax.axis_index("c")
    sid = lax.axis_index("s")
    wid = sid * NC + cid
    sbuf = [sb0, sb1, sb2, sb3]
    rbuf = [rb0, rb1, rb2, rb3]
    isem = [i0, i1, i2, i3]
    qsem = [q0, q1, q2, q3]

    zero16 = jnp.zeros((16,), jnp.float32)
    one16 = jnp.ones((16,), jnp.float32)

    @pl.loop(0, ROWS_T)
    def _(i):
        zbuf[i, :] = zero16

    @pl.loop(0, CHUNK)
    def _(i):
        ones_v[i, :] = one16

    sl = pl.ds(sid * ROWS_T, ROWS_T)
    pltpu.sync_copy(zbuf, accs.at[sl])
    pltpu.sync_copy(zbuf, accr.at[sl])
    plsc.subcore_barrier()

    def process(jbase, count):
        idescs = []
        for k in range(count):
            c = (jbase + k) * NW + wid
            d1 = pltpu.async_copy(s_ref.at[c], sbuf[k], isem[k])
            d2 = pltpu.async_copy(r_ref.at[c], rbuf[k], isem[k])
            idescs.append((d1, d2))
        sdescs = []
        for k in range(count):
            idescs[k][0].wait()
            idescs[k][1].wait()
            sdescs.append(pltpu.async_copy(ones_v, accs.at[sbuf[k].at[0]],
                                           qsem[k], add=True))
            sdescs.append(pltpu.async_copy(ones_v, accr.at[rbuf[k].at[0]],
                                           qsem[k], add=True))
        for d in sdescs:
            d.wait()

    @pl.loop(0, MAIN, step=NB)
    def _(J):
        process(J, NB)

    c_tail = MAIN * NW + wid

    @pl.when(c_tail < NCH)
    def _():
        pltpu.sync_copy(s_ref.at[c_tail], sbuf[0])
        pltpu.sync_copy(r_ref.at[c_tail], rbuf[0])
        pltpu.sync_copy(ones_v, accs.at[sbuf[0].at[0]], add=True)
        pltpu.sync_copy(ones_v, accr.at[rbuf[0].at[0]], add=True)

    plsc.subcore_barrier()
    pltpu.sync_copy(accs.at[sl], out_s.at[cid, sid])
    pltpu.sync_copy(accr.at[sl], out_r.at[cid, sid])


def _sc_hist(s2, r2):
    kern = pl.kernel(
        _hist_body,
        out_type=(jax.ShapeDtypeStruct((NC, NS, ROWS_T, 16), jnp.float32),
                  jax.ShapeDtypeStruct((NC, NS, ROWS_T, 16), jnp.float32)),
        mesh=_MESH,
        scratch_types=(
            [pltpu.VMEM((1, CHUNK), jnp.int32)] * 8
            + [pltpu.VMEM((CHUNK, 16), jnp.float32),
               pltpu.VMEM((ROWS_T, 16), jnp.float32),
               pltpu.VMEM_SHARED((NPAD, 16), jnp.float32),
               pltpu.VMEM_SHARED((NPAD, 16), jnp.float32)]
            + [pltpu.SemaphoreType.DMA] * 8),
        compiler_params=_SC_PARAMS,
    )
    hs, hr = kern(s2, r2)
    return hs.reshape(NC, N, 16), hr.reshape(NC, N, 16)


def _conv_body(xs_ref, s_ref, r_ref, out_ref,
               sidx0, sidx1, sidx2, ridx0, ridx1, ridx2,
               rows0, rows1, rows2, acc,
               g0, g1, g2, sm0, sm1, sm2, i0, i1, i2):
    cid = lax.axis_index("c")
    sid = lax.axis_index("s")
    wid = sid * NC + cid
    sidx = [sidx0, sidx1, sidx2]
    ridx = [ridx0, ridx1, ridx2]
    rows = [rows0, rows1, rows2]
    g = [g0, g1, g2]
    sm = [sm0, sm1, sm2]
    isem = [i0, i1, i2]

    zero16 = jnp.zeros((16,), jnp.float32)
    for lg in range(D // 16):
        @pl.loop(0, ZROWS)
        def _(i):
            rows0[i, lg * 16:(lg + 1) * 16] = zero16

    @pl.loop(0, ROWS_T // ZROWS)
    def _(kz):
        pltpu.sync_copy(rows0.at[pl.ds(0, ZROWS)],
                        acc.at[pl.ds(sid * ROWS_T + kz * ZROWS, ZROWS)])

    plsc.subcore_barrier()

    def process(jbase, count):
        idescs = []
        for k in range(count):
            c = (jbase + k) * NW + wid
            d1 = pltpu.async_copy(s_ref.at[c], sidx[k], isem[k])
            d2 = pltpu.async_copy(r_ref.at[c], ridx[k], isem[k])
            idescs.append((d1, d2))
        gdescs = [None] * count
        sdescs = [None] * count

        def start_gather(k):
            idescs[k][0].wait()
            idescs[k][1].wait()
            gdescs[k] = pltpu.async_copy(xs_ref.at[sidx[k].at[0]], rows[k],
                                         g[k])

        def start_scatter(k):
            gdescs[k].wait()
            sdescs[k] = pltpu.async_copy(rows[k], acc.at[ridx[k].at[0]],
                                         sm[k], add=True)

        start_gather(0)
        for k in range(1, count):
            start_gather(k)
            start_scatter(k - 1)
        start_scatter(count - 1)
        for k in range(count):
            sdescs[k].wait()

    @pl.loop(0, MAIN, step=NB)
    def _(J):
        process(J, NB)

    c_tail = MAIN * NW + wid

    @pl.when(c_tail < NCH)
    def _():
        pltpu.sync_copy(s_ref.at[c_tail], sidx[0])
        pltpu.async_copy(xs_ref.at[sidx[0].at[0]], rows[0], g[0]).wait()
        pltpu.sync_copy(r_ref.at[c_tail], ridx[0])
        pltpu.sync_copy(rows[0], acc.at[ridx[0].at[0]], add=True)

    plsc.subcore_barrier()
    sl = pl.ds(sid * ROWS_T, ROWS_T)
    pltpu.sync_copy(acc.at[sl], out_ref.at[cid, sid])


def _sc_conv(xs, s2, r2):
    kern = pl.kernel(
        _conv_body,
        out_type=jax.ShapeDtypeStruct((NC, NS, ROWS_T, D), jnp.float32),
        mesh=_MESH,
        scratch_types=(
            [pltpu.VMEM((1, CHUNK), jnp.int32)] * 6
            + [pltpu.VMEM((CHUNK, D), jnp.float32)] * 3
            + [pltpu.VMEM_SHARED((NPAD, D), jnp.float32)]
            + [pltpu.SemaphoreType.DMA] * 9),
        compiler_params=_SC_PARAMS,
    )
    return kern(xs, s2, r2).reshape(NC, N, D)


def _mlp(x, w0, b0, w1, b1, hs):
    x = jnp.maximum(jnp.dot(x, w0, preferred_element_type=jnp.float32) + b0,
                    0.0)
    x = jnp.maximum(jnp.dot(x, w1, preferred_element_type=jnp.float32) + b1,
                    0.0)
    sdeg = hs[0, :, 0:1] + hs[1, :, 0:1] + 1.0
    return x * lax.rsqrt(sdeg)


def _embed_body(nodes_ref, we_ref, be_ref, edges_ref, w0_ref, b0_ref,
                w1_ref, b1_ref, hs_ref, h_ref, e4_ref, xs_ref):
    h = (jnp.dot(nodes_ref[...], we_ref[...],
                 preferred_element_type=jnp.float32) + be_ref[...])
    h_ref[...] = h
    e4_ref[...] = edges_ref[...] * 4.0
    xs_ref[...] = _mlp(h, w0_ref[...], b0_ref[...], w1_ref[...], b1_ref[...],
                       hs_ref[...])


def _tc_embed_mlp(nodes, W_embed, b_embed, e2, w0, b0, w1, b1, hist_s):
    return pl.pallas_call(
        _embed_body,
        grid=(TCG,),
        in_specs=[
            pl.BlockSpec((TCB, D), lambda i: (i, 0)),
            pl.BlockSpec((D, D), lambda i: (0, 0)),
            pl.BlockSpec((D,), lambda i: (0,)),
            pl.BlockSpec((TCB, D), lambda i: (i, 0)),
            pl.BlockSpec((D, D), lambda i: (0, 0)),
            pl.BlockSpec((D,), lambda i: (0,)),
            pl.BlockSpec((D, D), lambda i: (0, 0)),
            pl.BlockSpec((D,), lambda i: (0,)),
            pl.BlockSpec((NC, TCB, 16), lambda i: (0, i, 0)),
        ],
        out_specs=[
            pl.BlockSpec((TCB, D), lambda i: (i, 0)),
            pl.BlockSpec((TCB, D), lambda i: (i, 0)),
            pl.BlockSpec((TCB, D), lambda i: (i, 0)),
        ],
        out_shape=[jax.ShapeDtypeStruct((N, D), jnp.float32),
                   jax.ShapeDtypeStruct((N, D), jnp.float32),
                   jax.ShapeDtypeStruct((N, D), jnp.float32)],
    )(nodes, W_embed, b_embed, e2, w0, b0, w1, b1, hist_s)


def _updated(acc, xs, h, hr, sc, bi):
    rdeg = hr[0, :, 0:1] + hr[1, :, 0:1] + 1.0
    t = (acc[0] + acc[1] + xs) * lax.rsqrt(rdeg) + h
    m = jnp.mean(t, axis=-1, keepdims=True)
    v = jnp.mean(jnp.square(t - m), axis=-1, keepdims=True)
    return ((t - m) * lax.rsqrt(v + 1e-6)) * sc + bi


def _update_mlp_body(acc_ref, xs_ref, h_ref, hr_ref, sc_ref, bi_ref,
                     w0_ref, b0_ref, w1_ref, b1_ref, hs_ref,
                     out_ref, xs2_ref):
    hn = _updated(acc_ref[...], xs_ref[...], h_ref[...], hr_ref[...],
                  sc_ref[...], bi_ref[...])
    out_ref[...] = hn
    xs2_ref[...] = _mlp(hn, w0_ref[...], b0_ref[...], w1_ref[...],
                        b1_ref[...], hs_ref[...])


def _tc_update_mlp(acc, xs, h, hist_r, lns, lnb, w0, b0, w1, b1, hist_s):
    return pl.pallas_call(
        _update_mlp_body,
        grid=(TCG,),
        in_specs=[
            pl.BlockSpec((NC, TCB, D), lambda i: (0, i, 0)),
            pl.BlockSpec((TCB, D), lambda i: (i, 0)),
            pl.BlockSpec((TCB, D), lambda i: (i, 0)),
            pl.BlockSpec((NC, TCB, 16), lambda i: (0, i, 0)),
            pl.BlockSpec((D,), lambda i: (0,)),
            pl.BlockSpec((D,), lambda i: (0,)),
            pl.BlockSpec((D, D), lambda i: (0, 0)),
            pl.BlockSpec((D,), lambda i: (0,)),
            pl.BlockSpec((D, D), lambda i: (0, 0)),
            pl.BlockSpec((D,), lambda i: (0,)),
            pl.BlockSpec((NC, TCB, 16), lambda i: (0, i, 0)),
        ],
        out_specs=[
            pl.BlockSpec((TCB, D), lambda i: (i, 0)),
            pl.BlockSpec((TCB, D), lambda i: (i, 0)),
        ],
        out_shape=[jax.ShapeDtypeStruct((N, D), jnp.float32),
                   jax.ShapeDtypeStruct((N, D), jnp.float32)],
    )(acc, xs, h, hist_r, lns, lnb, w0, b0, w1, b1, hist_s)


def _update_body(acc_ref, xs_ref, h_ref, hr_ref, sc_ref, bi_ref,
                 wd_ref, bd_ref, out_ref, outg_ref, pooled_ref):
    i = pl.program_id(0)
    hn = _updated(acc_ref[...], xs_ref[...], h_ref[...],
                  hr_ref[...], sc_ref[...], bi_ref[...])
    out_ref[...] = hn
    col_graph = (lax.broadcasted_iota(jnp.int32, (NG, TCB), 1)
                 + i * TCB) // ROWS_G
    row_id = lax.broadcasted_iota(jnp.int32, (NG, TCB), 0)
    gsel = (col_graph == row_id).astype(jnp.float32)
    part = jnp.dot(gsel, hn, preferred_element_type=jnp.float32)

    @pl.when(i == 0)
    def _():
        pooled_ref[...] = jnp.zeros((NG, D), jnp.float32)

    pooled_ref[...] += part

    @pl.when(i == TCG - 1)
    def _():
        pooled = pooled_ref[...] * (1.0 / ROWS_G)
        outg_ref[...] = (jnp.dot(pooled, wd_ref[...],
                                 preferred_element_type=jnp.float32)
                         + bd_ref[...])


def _tc_update_decode(acc, xs, h, hist_r, lns, lnb, W_dec, b_dec):
    return pl.pallas_call(
        _update_body,
        grid=(TCG,),
        in_specs=[
            pl.BlockSpec((NC, TCB, D), lambda i: (0, i, 0)),
            pl.BlockSpec((TCB, D), lambda i: (i, 0)),
            pl.BlockSpec((TCB, D), lambda i: (i, 0)),
            pl.BlockSpec((NC, TCB, 16), lambda i: (0, i, 0)),
            pl.BlockSpec((D,), lambda i: (0,)),
            pl.BlockSpec((D,), lambda i: (0,)),
            pl.BlockSpec((D, OUT_G), lambda i: (0, 0)),
            pl.BlockSpec((OUT_G,), lambda i: (0,)),
        ],
        out_specs=[
            pl.BlockSpec((TCB, D), lambda i: (i, 0)),
            pl.BlockSpec((NG, OUT_G), lambda i: (0, 0)),
        ],
        out_shape=[jax.ShapeDtypeStruct((N, D), jnp.float32),
                   jax.ShapeDtypeStruct((NG, OUT_G), jnp.float32)],
        scratch_shapes=[pltpu.VMEM((NG, D), jnp.float32)],
    )(acc, xs, h, hist_r, lns, lnb, W_dec, b_dec)


def kernel(nodes, edges, senders, receivers, globals_, n_node, n_edge,
           W_embed, b_embed,
           W_s0_l0, b_s0_l0, W_s0_l1, b_s0_l1, ln0_scale, ln0_bias,
           W_s1_l0, b_s1_l0, W_s1_l1, b_s1_l1, ln1_scale, ln1_bias,
           W_dec, b_dec):
    s2 = senders.reshape(NCH, 1, CHUNK)
    r2 = receivers.reshape(NCH, 1, CHUNK)
    e2 = edges.reshape(N, D)

    hist_s, hist_r = _sc_hist(s2, r2)
    h, e4, xs = _tc_embed_mlp(nodes, W_embed, b_embed, e2,
                              W_s0_l0, b_s0_l0, W_s0_l1, b_s0_l1, hist_s)
    acc = _sc_conv(xs, s2, r2)
    h, xs = _tc_update_mlp(acc, xs, h, hist_r, ln0_scale, ln0_bias,
                           W_s1_l0, b_s1_l0, W_s1_l1, b_s1_l1, hist_s)
    acc = _sc_conv(xs, s2, r2)
    h, out_globals = _tc_update_decode(acc, xs, h, hist_r,
                                       ln1_scale, ln1_bias, W_dec, b_dec)
    return h, e4.reshape(E, 4), out_globals
